# Initial kernel scaffold; baseline (speedup 1.0000x reference)
import jax
import jax.numpy as jnp
from jax.experimental import pallas as pl


def kernel(logits, T, Qhat, penalties):
    raise NotImplementedError("write your pallas kernel here")


# speedup vs baseline: 173.1544x; 173.1544x over previous
import functools

import jax
import jax.numpy as jnp
from jax import lax
from jax.experimental import pallas as pl
from jax.experimental.pallas import tpu as pltpu
from jax.experimental.pallas import tpu_sc as plsc

BATCH = 128
V = 100000
CHUNK = 1024
NCHUNK = 98
VP = NCHUNK * CHUNK
BSZ = 128
NBLK = VP // BSZ
KSEL = 112
GMAX = 128
CBUF = 256
NCOND = 112
NEG = -3.4e38


def _stage_a(x_ref, t_ref, xpad_ref, den_ref, bm_ref):
    j = pl.program_id(0)
    col = j * CHUNK + lax.broadcasted_iota(jnp.int32, (BATCH, CHUNK), 1)
    x = jnp.where(col < V, x_ref[...], NEG)
    xpad_ref[...] = x
    e = jnp.exp(x / t_ref[0, 0])
    part = jnp.sum(e, axis=1, keepdims=True)

    @pl.when(j == 0)
    def _():
        den_ref[...] = jnp.zeros_like(den_ref)

    den_ref[...] = den_ref[...] + jnp.broadcast_to(part, (BATCH, 16))
    bm_ref[...] = jnp.concatenate(
        [jnp.max(x[:, s * BSZ:(s + 1) * BSZ], axis=1, keepdims=True)
         for s in range(CHUNK // BSZ)], axis=1)[None]


def _stage_b(bm_ref, t_ref):
    bm = bm_ref[...]
    u = lax.bitcast_convert_type(bm, jnp.uint32)
    us = jnp.where(u >> 31 == 1, ~u, u | jnp.uint32(0x80000000))
    prefix = jnp.zeros((BATCH, 1), jnp.uint32)
    for b in range(31, -1, -1):
        cand = prefix | jnp.uint32(1 << b)
        cnt = jnp.sum((us >= cand).astype(jnp.int32), axis=1, keepdims=True)
        prefix = jnp.where(cnt >= KSEL, cand, prefix)
    uu = jnp.where(prefix >> 31 == 1, prefix ^ jnp.uint32(0x80000000), ~prefix)
    tf = lax.bitcast_convert_type(uu, jnp.float32)
    t_ref[...] = jnp.broadcast_to(tf, (BATCH, 16))


def _cx(a, b):
    c = a[0] >= b[0]
    return ((jnp.where(c, a[0], b[0]), jnp.where(c, a[1], b[1])),
            (jnp.where(c, b[0], a[0]), jnp.where(c, b[1], a[1])))


def _vsort(p):
    k, v = plsc.sort_key_val(p[0], p[1], descending=True)
    return (k, v)


def _merge(a, b):
    m = len(a)
    c = a + [(lax.rev(x[0], (0,)), lax.rev(x[1], (0,))) for x in reversed(b)]
    d = m
    while d >= 1:
        for i0 in range(0, 2 * m, 2 * d):
            for i in range(i0, i0 + d):
                c[i], c[i + d] = _cx(c[i], c[i + d])
        d //= 2
    return [_vsort(p) for p in c]


def _sc_stage(xblk, bm_h, t_h, den_h, pen_h, sca_h,
              sizes_o, thr_o, ccut_o,
              bm_v, t_v, den_v, pen_v, sca_v,
              bid_v, gidx_v, gbuf_v, cval_v, cpos_v, sval_v, spos_v, tie_v,
              osz_v, othr_v, occ_v, sem):
    wid = lax.axis_index("s") * 2 + lax.axis_index("c")
    iota16 = lax.iota(jnp.int32, 16)
    zeros16 = jnp.zeros((16,), jnp.int32)

    pltpu.sync_copy(pen_h, pen_v)
    pltpu.sync_copy(sca_h, sca_v)
    sca = sca_v[...]
    qhat = jnp.sum(sca * jnp.where(iota16 == 0, 1.0, 0.0))
    tdiv = jnp.sum(sca * jnp.where(iota16 == 1, 1.0, 0.0))

    pencum = []
    carry = jnp.float32(0.0)
    for k in range(NCOND // 16):
        ch = pen_v[pl.ds(16 * k, 16)]
        pencum.append(plsc.cumsum(ch) + carry)
        carry = carry + jnp.sum(ch)

    def row_body(j, outs):
        osz, othr, occ = outs
        r = wid * 4 + j
        pltpu.sync_copy(bm_h.at[r], bm_v)
        pltpu.sync_copy(t_h.at[r], t_v)
        pltpu.sync_copy(den_h.at[r], den_v)
        ts = jnp.max(t_v[...])
        den = jnp.max(den_v[...])

        for k in range(GMAX // 16 + 1):
            bid_v[pl.ds(16 * k, 16)] = iota16
        nb = jnp.int32(0)
        for k in range(NBLK // 16):
            m = bm_v[pl.ds(16 * k, 16)] >= ts
            m = m & (nb < GMAX)
            dst = nb + plsc.cumsum(jnp.where(m, 1, 0)) - 1
            plsc.store_scatter(bid_v, [dst], iota16 + 16 * k, mask=m)
            nb = nb + jnp.sum(jnp.where(m, 1, 0))
        nb_used = jnp.minimum(nb, GMAX)

        for k in range(GMAX // 16):
            gidx_v[pl.ds(16 * k, 16)] = bid_v[pl.ds(16 * k, 16)] + r * NBLK
        pltpu.async_copy(xblk.at[gidx_v], gbuf_v, sem).wait()

        for k in range(CBUF // 16 + 1):
            cval_v[pl.ds(16 * k, 16)] = jnp.full((16,), NEG, jnp.float32)
            cpos_v[pl.ds(16 * k, 16)] = zeros16 + jnp.int32(0x7FFFFF)

        def fbody(g, w):
            for k in range(BSZ // 16):
                v = gbuf_v[g, pl.ds(16 * k, 16)]
                m = (v >= ts) & (w < CBUF)
                dst = w + plsc.cumsum(jnp.where(m, 1, 0)) - 1
                pos = g * BSZ + 16 * k + iota16
                plsc.store_scatter(cval_v, [dst], v, mask=m)
                plsc.store_scatter(cpos_v, [dst], pos, mask=m)
                w = w + jnp.sum(jnp.where(m, 1, 0))
            return w

        lax.fori_loop(0, nb_used, fbody, jnp.int32(0))

        lists = [[_vsort((cval_v[pl.ds(16 * k, 16)], cpos_v[pl.ds(16 * k, 16)]))]
                 for k in range(CBUF // 16)]
        while len(lists) > 1:
            lists = [_merge(lists[i], lists[i + 1])
                     for i in range(0, len(lists), 2)]
        srt = lists[0]
        for k in range(CBUF // 16):
            sval_v[pl.ds(16 * k, 16)] = srt[k][0]
            spos_v[pl.ds(16 * k, 16)] = srt[k][1]

        cnt = jnp.int32(0)
        cc = jnp.float32(0.0)
        for k in range(NCOND // 16):
            sc = jnp.exp(srt[k][0] / tdiv) / den
            cs = plsc.cumsum(sc) + cc
            cc = cc + jnp.sum(sc)
            cond = (cs + pencum[k]) <= qhat
            cnt = cnt + jnp.sum(jnp.where(cond, 1, 0))
        size = jnp.minimum(cnt + 1, V)

        thr = jnp.max(plsc.load_gather(sval_v, [zeros16 + cnt]))
        cgt = jnp.int32(0)
        for k in range(CBUF // 16):
            cgt = cgt + jnp.sum(jnp.where(srt[k][0] > thr, 1, 0))
        q = size - cgt

        tie_v[pl.ds(0, 16)] = zeros16 + jnp.int32(0x7FFFFFFF)
        tie_v[pl.ds(16, 16)] = zeros16 + jnp.int32(0x7FFFFFFF)
        tw = jnp.int32(0)
        for k in range(CBUF // 16):
            m = (srt[k][0] == thr) & (tw < 16)
            dst = tw + plsc.cumsum(jnp.where(m, 1, 0)) - 1
            plsc.store_scatter(tie_v, [dst], srt[k][1], mask=m)
            tw = tw + jnp.sum(jnp.where(m, 1, 0))
        tsort, _ = plsc.sort_key_val(tie_v[pl.ds(0, 16)],
                                     tie_v[pl.ds(0, 16)], descending=False)
        tie_v[pl.ds(0, 16)] = tsort
        cpos = jnp.max(plsc.load_gather(tie_v, [zeros16 + (q - 1)]))
        g = lax.shift_right_logical(cpos, 7)
        off = cpos & jnp.int32(BSZ - 1)
        bid = jnp.max(plsc.load_gather(bid_v, [zeros16 + g]))
        ccls = bid * BSZ + off

        sel = iota16 == j
        return (jnp.where(sel, size, osz),
                jnp.where(sel, thr, othr),
                jnp.where(sel, ccls, occ))

    osz, othr, occ = lax.fori_loop(
        0, 4, row_body,
        (zeros16, jnp.zeros((16,), jnp.float32), zeros16))
    osz_v[...] = osz
    othr_v[...] = othr
    occ_v[...] = occ
    pltpu.sync_copy(osz_v, sizes_o.at[wid])
    pltpu.sync_copy(othr_v, thr_o.at[wid])
    pltpu.sync_copy(occ_v, ccut_o.at[wid])


def _stage_d(x_ref, thr_ref, cc_ref, mask_ref):
    j = pl.program_id(0)
    x = x_ref[...]
    thr = thr_ref[:, 0:1]
    cc = cc_ref[:, 0:1]
    col = j * CHUNK + lax.broadcasted_iota(jnp.int32, (BATCH, CHUNK), 1)
    mask_ref[...] = (x > thr) | ((x == thr) & (col <= cc))


def kernel(logits, T, Qhat, penalties):
    t2d = T.reshape(1, 1)
    xpad, den, bm = pl.pallas_call(
        _stage_a,
        grid=(NCHUNK,),
        in_specs=[
            pl.BlockSpec((BATCH, CHUNK), lambda j: (0, j)),
            pl.BlockSpec((1, 1), lambda j: (0, 0)),
        ],
        out_specs=[
            pl.BlockSpec((BATCH, CHUNK), lambda j: (0, j)),
            pl.BlockSpec((BATCH, 16), lambda j: (0, 0)),
            pl.BlockSpec((1, BATCH, CHUNK // BSZ), lambda j: (j, 0, 0)),
        ],
        out_shape=[
            jax.ShapeDtypeStruct((BATCH, VP), jnp.float32),
            jax.ShapeDtypeStruct((BATCH, 16), jnp.float32),
            jax.ShapeDtypeStruct((NCHUNK, BATCH, CHUNK // BSZ), jnp.float32),
        ],
        compiler_params=pltpu.CompilerParams(
            dimension_semantics=("arbitrary",)),
    )(logits, t2d)
    bm = bm.transpose(1, 0, 2).reshape(BATCH, NBLK)

    tthr = pl.pallas_call(
        _stage_b,
        in_specs=[pl.BlockSpec((BATCH, NBLK), lambda: (0, 0))],
        out_specs=pl.BlockSpec((BATCH, 16), lambda: (0, 0)),
        out_shape=jax.ShapeDtypeStruct((BATCH, 16), jnp.float32),
    )(bm)

    xblk = xpad.reshape(BATCH * NBLK, BSZ)
    pen112 = penalties[0, :NCOND]
    sca = jnp.zeros((16,), jnp.float32).at[0].set(Qhat).at[1].set(T)

    mesh = plsc.VectorSubcoreMesh(core_axis_name="c", subcore_axis_name="s")
    sizes32, thr32, ccut32 = pl.kernel(
        _sc_stage,
        mesh=mesh,
        compiler_params=pltpu.CompilerParams(needs_layout_passes=False),
        out_type=[
            jax.ShapeDtypeStruct((32, 16), jnp.int32),
            jax.ShapeDtypeStruct((32, 16), jnp.float32),
            jax.ShapeDtypeStruct((32, 16), jnp.int32),
        ],
        scratch_types=[
            pltpu.VMEM((NBLK,), jnp.float32),
            pltpu.VMEM((16,), jnp.float32),
            pltpu.VMEM((16,), jnp.float32),
            pltpu.VMEM((NCOND,), jnp.float32),
            pltpu.VMEM((16,), jnp.float32),
            pltpu.VMEM((GMAX + 16,), jnp.int32),
            pltpu.VMEM((GMAX,), jnp.int32),
            pltpu.VMEM((GMAX, BSZ), jnp.float32),
            pltpu.VMEM((CBUF + 16,), jnp.float32),
            pltpu.VMEM((CBUF + 16,), jnp.int32),
            pltpu.VMEM((CBUF,), jnp.float32),
            pltpu.VMEM((CBUF,), jnp.int32),
            pltpu.VMEM((32,), jnp.int32),
            pltpu.VMEM((16,), jnp.int32),
            pltpu.VMEM((16,), jnp.float32),
            pltpu.VMEM((16,), jnp.int32),
            pltpu.SemaphoreType.DMA,
        ],
    )(xblk, bm, tthr, den, pen112, sca)

    sizes = sizes32[:, :4].reshape(BATCH)
    thr = jnp.broadcast_to(thr32[:, :4].reshape(BATCH, 1), (BATCH, 16))
    ccut = jnp.broadcast_to(ccut32[:, :4].reshape(BATCH, 1), (BATCH, 16))

    mask = pl.pallas_call(
        _stage_d,
        grid=(NCHUNK,),
        in_specs=[
            pl.BlockSpec((BATCH, CHUNK), lambda j: (0, j)),
            pl.BlockSpec((BATCH, 16), lambda j: (0, 0)),
            pl.BlockSpec((BATCH, 16), lambda j: (0, 0)),
        ],
        out_specs=pl.BlockSpec((BATCH, CHUNK), lambda j: (0, j)),
        out_shape=jax.ShapeDtypeStruct((BATCH, V), jnp.bool_),
        compiler_params=pltpu.CompilerParams(
            dimension_semantics=("arbitrary",)),
    )(logits, thr, ccut)

    return logits, sizes, mask
